# Initial kernel scaffold; baseline (speedup 1.0000x reference)
import jax
import jax.numpy as jnp
from jax import lax
from jax.experimental import pallas as pl
from jax.experimental.pallas import tpu as pltpu
from jax.experimental.pallas import tpu_sc as plsc

B, L = 4096, 200
D = 64
N = B * L
NC, NS = 2, 16
NW = NC * NS
TOK_PER_W = N // NW
GATHER = 128
KSUB = 4
SUP = GATHER * KSUB
ITERS = TOK_PER_W // SUP
ROWS_IDX = N // GATHER


def _prep_kernel(type_ref, hyp_ref, pos_ref, typ_ids_ref, hyp_ids_ref,
                 pos_ids_ref, comb_ref, cidx_ref):
  comb_ref[...] = (
      type_ref[...][:, None, None, :]
      + hyp_ref[...][None, :, None, :]
      + pos_ref[...][None, None, :, :]
  )
  cidx_ref[...] = (
      typ_ids_ref[...] * 1024 + hyp_ids_ref[...] * 512 + pos_ids_ref[...]
  )


def _sc_body(tok_hbm, cidx_hbm, emb_hbm, comb_hbm, out_hbm,
             tok_v, cidx_v, emb_rows, comb_rows, sem_e, sem_c):
  w = lax.axis_index("s") * NC + lax.axis_index("c")
  row0 = w * (TOK_PER_W // GATHER)

  def body(i, carry):
    base_row = row0 + i * KSUB
    base_tok = base_row * GATHER
    pltpu.sync_copy(tok_hbm.at[pl.ds(base_row, KSUB)], tok_v)
    pltpu.sync_copy(cidx_hbm.at[pl.ds(base_row, KSUB)], cidx_v)
    copies = []
    for j in range(KSUB):
      copies.append(pltpu.async_copy(
          emb_hbm.at[tok_v.at[j]],
          emb_rows.at[pl.ds(j * GATHER, GATHER)], sem_e))
      copies.append(pltpu.async_copy(
          comb_hbm.at[cidx_v.at[j]],
          comb_rows.at[pl.ds(j * GATHER, GATHER)], sem_c))
    for c in copies:
      c.wait()

    def add_row(r, carry2):
      for g in range(D // 16):
        sl = pl.ds(g * 16, 16)
        emb_rows[r, sl] = emb_rows[r, sl] + comb_rows[r, sl]
      return carry2

    lax.fori_loop(0, SUP, add_row, 0, unroll=2)
    pltpu.sync_copy(emb_rows, out_hbm.at[pl.ds(base_tok, SUP)])
    return carry

  lax.fori_loop(0, ITERS, body, 0)


def kernel(input_ids, token_type_ids, position_ids, if_hyp_ids,
           embed_table, type_table, pos_table, hyp_table):
  tok = input_ids.astype(jnp.int32).reshape(ROWS_IDX, GATHER)
  typ = token_type_ids.astype(jnp.int32).reshape(ROWS_IDX, GATHER)
  pos = position_ids.astype(jnp.int32).reshape(ROWS_IDX, GATHER)
  hyp = if_hyp_ids.astype(jnp.int32).reshape(ROWS_IDX, GATHER)

  comb4, cidx = pl.pallas_call(
      _prep_kernel,
      out_shape=(
          jax.ShapeDtypeStruct((2, 2, 512, D), jnp.float32),
          jax.ShapeDtypeStruct((ROWS_IDX, GATHER), jnp.int32),
      ),
  )(type_table, hyp_table, pos_table, typ, hyp, pos)
  comb = comb4.reshape(2 * 2 * 512, D)

  mesh = plsc.VectorSubcoreMesh(core_axis_name="c", subcore_axis_name="s")
  sc = pl.kernel(
      _sc_body,
      out_type=jax.ShapeDtypeStruct((N, D), jnp.float32),
      mesh=mesh,
      scratch_types=[
          pltpu.VMEM((KSUB, GATHER), jnp.int32),
          pltpu.VMEM((KSUB, GATHER), jnp.int32),
          pltpu.VMEM((SUP, D), jnp.float32),
          pltpu.VMEM((SUP, D), jnp.float32),
          pltpu.SemaphoreType.DMA,
          pltpu.SemaphoreType.DMA,
      ],
  )
  out = sc(tok, cidx, embed_table, comb)
  return out.reshape(B, L, D)


# speedup vs baseline: 8.3309x; 8.3309x over previous
import jax
import jax.numpy as jnp
from jax import lax
from jax.experimental import pallas as pl
from jax.experimental.pallas import tpu as pltpu
from jax.experimental.pallas import tpu_sc as plsc

B, L = 4096, 200
D = 64
N = B * L
NC, NS = 2, 16
NW = NC * NS
TOK_PER_W = N // NW
GATHER = 128
KSUB = 4
SUP = GATHER * KSUB
ITERS = TOK_PER_W // SUP
ROWS_IDX = N // GATHER


def _prep_kernel(type_ref, hyp_ref, pos_ref, typ_ids_ref, hyp_ids_ref,
                 pos_ids_ref, comb_ref, cidx_ref):
  comb_ref[...] = (
      type_ref[...][:, None, None, :]
      + hyp_ref[...][None, :, None, :]
      + pos_ref[...][None, None, :, :]
  )
  cidx_ref[...] = (
      typ_ids_ref[...] * 1024 + hyp_ids_ref[...] * 512 + pos_ids_ref[...]
  )


def _sc_body(tok_hbm, cidx_hbm, emb_hbm, comb_hbm, out_hbm,
             tok_v, cidx_v, emb_rows, comb_rows, sem_e, sem_c):
  w = lax.axis_index("s") * NC + lax.axis_index("c")
  row0 = w * (TOK_PER_W // GATHER)

  def body(i, carry):
    base_row = row0 + i * KSUB
    base_tok = base_row * GATHER
    pltpu.sync_copy(tok_hbm.at[pl.ds(base_row, KSUB)], tok_v)
    pltpu.sync_copy(cidx_hbm.at[pl.ds(base_row, KSUB)], cidx_v)
    copies = []
    for j in range(KSUB):
      copies.append(pltpu.async_copy(
          emb_hbm.at[tok_v.at[j]],
          emb_rows.at[pl.ds(j * GATHER, GATHER)], sem_e))
      copies.append(pltpu.async_copy(
          comb_hbm.at[cidx_v.at[j]],
          comb_rows.at[pl.ds(j * GATHER, GATHER)], sem_c))
    for c in copies:
      c.wait()

    def add_row(r, carry2):
      for g in range(D // 16):
        sl = pl.ds(g * 16, 16)
        emb_rows[r, sl] = emb_rows[r, sl] + comb_rows[r, sl]
      return carry2

    lax.fori_loop(0, SUP, add_row, 0, unroll=2)
    pltpu.sync_copy(emb_rows, out_hbm.at[pl.ds(base_tok, SUP)])
    return carry

  lax.fori_loop(0, ITERS, body, 0)


def kernel(input_ids, token_type_ids, position_ids, if_hyp_ids,
           embed_table, type_table, pos_table, hyp_table):
  tok = input_ids.astype(jnp.int32).reshape(ROWS_IDX, GATHER)
  typ = token_type_ids.astype(jnp.int32).reshape(ROWS_IDX, GATHER)
  pos = position_ids.astype(jnp.int32).reshape(ROWS_IDX, GATHER)
  hyp = if_hyp_ids.astype(jnp.int32).reshape(ROWS_IDX, GATHER)

  comb4, cidx = pl.pallas_call(
      _prep_kernel,
      out_shape=(
          jax.ShapeDtypeStruct((2, 2, 512, D), jnp.float32),
          jax.ShapeDtypeStruct((ROWS_IDX, GATHER), jnp.int32),
      ),
  )(type_table, hyp_table, pos_table, typ, hyp, pos)
  comb = comb4.reshape(2 * 2 * 512, D)

  mesh = plsc.VectorSubcoreMesh(core_axis_name="c", subcore_axis_name="s")
  sc = pl.kernel(
      _sc_body,
      out_type=jax.ShapeDtypeStruct((N, D), jnp.float32),
      mesh=mesh,
      compiler_params=pltpu.CompilerParams(use_tc_tiling_on_sc=False),
      scratch_types=[
          pltpu.VMEM((KSUB, GATHER), jnp.int32),
          pltpu.VMEM((KSUB, GATHER), jnp.int32),
          pltpu.VMEM((SUP, D), jnp.float32),
          pltpu.VMEM((SUP, D), jnp.float32),
          pltpu.SemaphoreType.DMA,
          pltpu.SemaphoreType.DMA,
      ],
  )
  out = sc(tok, cidx, embed_table, comb)
  return out.reshape(B, L, D)


# speedup vs baseline: 10.8275x; 1.2997x over previous
import jax
import jax.numpy as jnp
from jax import lax
from jax.experimental import pallas as pl
from jax.experimental.pallas import tpu as pltpu
from jax.experimental.pallas import tpu_sc as plsc

B, L = 4096, 200
D = 64
N = B * L
NC, NS = 2, 16
NW = NC * NS
TOK_PER_W = N // NW
GATHER = 128
KSUB = 2
SUP = GATHER * KSUB
ITERS = TOK_PER_W // SUP
ROWS_IDX = N // GATHER


def _prep_kernel(type_ref, hyp_ref, pos_ref, typ_ids_ref, hyp_ids_ref,
                 pos_ids_ref, comb_ref, cidx_ref):
  comb_ref[...] = (
      type_ref[...][:, None, None, :]
      + hyp_ref[...][None, :, None, :]
      + pos_ref[...][None, None, :, :]
  )
  cidx_ref[...] = (
      typ_ids_ref[...] * 1024 + hyp_ids_ref[...] * 512 + pos_ids_ref[...]
  )


def _sc_body(tok_hbm, cidx_hbm, emb_hbm, comb_hbm, out_hbm,
             tok_v, cidx_v, emb_rows, comb_rows, out_rows,
             semg0, semg1, semo0, semo1):
  w = lax.axis_index("s") * NC + lax.axis_index("c")
  row0 = w * (TOK_PER_W // GATHER)
  semg = (semg0, semg1)
  semo = (semo0, semo1)

  def issue_gathers(i, b):
    pltpu.sync_copy(tok_hbm.at[pl.ds(row0 + i * KSUB, KSUB)], tok_v.at[b])
    pltpu.sync_copy(cidx_hbm.at[pl.ds(row0 + i * KSUB, KSUB)], cidx_v.at[b])
    for j in range(KSUB):
      dst = pl.ds(j * GATHER, GATHER)
      pltpu.async_copy(emb_hbm.at[tok_v.at[b, j]],
                       emb_rows.at[b].at[dst], semg[b])
      pltpu.async_copy(comb_hbm.at[cidx_v.at[b, j]],
                       comb_rows.at[b].at[dst], semg[b])

  def wait_gathers(i, b):
    for j in range(KSUB):
      dst = pl.ds(j * GATHER, GATHER)
      pltpu.make_async_copy(emb_hbm.at[tok_v.at[b, j]],
                            emb_rows.at[b].at[dst], semg[b]).wait()
      pltpu.make_async_copy(comb_hbm.at[cidx_v.at[b, j]],
                            comb_rows.at[b].at[dst], semg[b]).wait()

  def store(i, b):
    base_tok = (row0 + i * KSUB) * GATHER
    return pltpu.make_async_copy(out_rows.at[b],
                                 out_hbm.at[pl.ds(base_tok, SUP)], semo[b])

  issue_gathers(0, 0)

  def body(g, carry):
    for b in range(2):
      i = g * 2 + b
      nb = 1 - b

      @pl.when(i < ITERS - 1)
      def _():
        issue_gathers(i + 1, nb)

      wait_gathers(i, b)

      @pl.when(i >= 2)
      def _():
        store(i - 2, b).wait()

      def add_row(r, carry2):
        for gg in range(D // 16):
          sl = pl.ds(gg * 16, 16)
          out_rows[b, r, sl] = emb_rows[b, r, sl] + comb_rows[b, r, sl]
        return carry2

      lax.fori_loop(0, SUP, add_row, 0, unroll=2)
      store(i, b).start()
    return carry

  lax.fori_loop(0, ITERS // 2, body, 0)
  store(ITERS - 2, 0).wait()
  store(ITERS - 1, 1).wait()


def kernel(input_ids, token_type_ids, position_ids, if_hyp_ids,
           embed_table, type_table, pos_table, hyp_table):
  tok = input_ids.astype(jnp.int32).reshape(ROWS_IDX, GATHER)
  typ = token_type_ids.astype(jnp.int32).reshape(ROWS_IDX, GATHER)
  pos = position_ids.astype(jnp.int32).reshape(ROWS_IDX, GATHER)
  hyp = if_hyp_ids.astype(jnp.int32).reshape(ROWS_IDX, GATHER)

  comb4, cidx = pl.pallas_call(
      _prep_kernel,
      out_shape=(
          jax.ShapeDtypeStruct((2, 2, 512, D), jnp.float32),
          jax.ShapeDtypeStruct((ROWS_IDX, GATHER), jnp.int32),
      ),
  )(type_table, hyp_table, pos_table, typ, hyp, pos)
  comb = comb4.reshape(2 * 2 * 512, D)

  mesh = plsc.VectorSubcoreMesh(core_axis_name="c", subcore_axis_name="s")
  sc = pl.kernel(
      _sc_body,
      out_type=jax.ShapeDtypeStruct((N, D), jnp.float32),
      mesh=mesh,
      compiler_params=pltpu.CompilerParams(use_tc_tiling_on_sc=False),
      scratch_types=[
          pltpu.VMEM((2, KSUB, GATHER), jnp.int32),
          pltpu.VMEM((2, KSUB, GATHER), jnp.int32),
          pltpu.VMEM((2, SUP, D), jnp.float32),
          pltpu.VMEM((2, SUP, D), jnp.float32),
          pltpu.VMEM((2, SUP, D), jnp.float32),
          pltpu.SemaphoreType.DMA,
          pltpu.SemaphoreType.DMA,
          pltpu.SemaphoreType.DMA,
          pltpu.SemaphoreType.DMA,
      ],
  )
  out = sc(tok, cidx, embed_table, comb)
  return out.reshape(B, L, D)
